# Initial kernel scaffold; baseline (speedup 1.0000x reference)
import jax
import jax.numpy as jnp
from jax.experimental import pallas as pl


def kernel(scores):
    raise NotImplementedError("write your pallas kernel here")


# speedup vs baseline: 80.1202x; 80.1202x over previous
import functools

import jax
import jax.numpy as jnp
from jax.experimental import pallas as pl


def _softargmax_block(scores_ref, out_ref, *, D):
    x = scores_ref[0]
    m = jnp.max(x, axis=0, keepdims=True)
    e = jnp.exp(x - m)
    d = jax.lax.broadcasted_iota(jnp.int32, e.shape, 0).astype(jnp.float32)
    num = jnp.sum(e * d, axis=0)
    den = jnp.sum(e, axis=0)
    out_ref[0, 0] = num / den


def kernel(scores):
    B, D, H, W = scores.shape
    hblk = 16
    grid = (B, H // hblk)
    out = pl.pallas_call(
        functools.partial(_softargmax_block, D=D),
        grid=grid,
        in_specs=[
            pl.BlockSpec((1, D, hblk, W), lambda b, h: (b, 0, h, 0)),
        ],
        out_specs=pl.BlockSpec((1, 1, hblk, W), lambda b, h: (b, 0, h, 0)),
        out_shape=jax.ShapeDtypeStruct((B, 1, H, W), scores.dtype),
    )(scores)
    return out
